# baseline (speedup 1.0000x reference)
import numpy as np
import jax
import jax.numpy as jnp
from jax import lax
from jax.experimental import pallas as pl
from jax.experimental.pallas import tpu as pltpu
from jax.experimental.pallas import tpu_sc as plsc

N = 10000
E = 320000
H = 256
HH = H // 2
L = 12

RN = 1000
RE = 1000

NSUB = 16
EPS = E // NSUB
CE = 32
CHUNKS = EPS // CE
SBC = 25
SBE = SBC * CE
CER = CE // 2
N_PAD = 10240
NPS = N_PAD // NSUB

_f32 = jnp.float32
_bf16 = jnp.bfloat16
_i32 = jnp.int32

_P = np.concatenate(
    [np.concatenate([32 * g + 2 * np.arange(16),
                     32 * g + 1 + 2 * np.arange(16)]) for g in range(H // 32)]
).astype(np.int32)
_INV_P = np.argsort(_P).astype(np.int32)


def _ln_relu(t, g, b):
    mu = jnp.mean(t, axis=-1, keepdims=True)
    var = jnp.mean((t - mu) ** 2, axis=-1, keepdims=True)
    return jnp.maximum((t - mu) * jax.lax.rsqrt(var + 1e-5) * g + b, 0.0)


def _split_tables(y, av_ref, b_ref):
    av_ref[0] = jnp.concatenate([y[:, 2 * H:2 * H + HH],
                                 y[:, 1 * H:1 * H + HH]], axis=-1).astype(_bf16)
    av_ref[1] = jnp.concatenate([y[:, 2 * H + HH:3 * H],
                                 y[:, 1 * H + HH:2 * H]], axis=-1).astype(_bf16)
    b_ref[0] = y[:, 3 * H:3 * H + HH]
    b_ref[1] = y[:, 3 * H + HH:4 * H]


def _node0_body(x_ref, nW_ref, nb_ref, Wc_ref, bc_ref,
                h_ref, u_ref, av_ref, b_ref):
    h = x_ref[...] @ nW_ref[...] + nb_ref[...]
    y = h @ Wc_ref[...] + bc_ref[...]
    h_ref[...] = h
    u_ref[...] = y[:, 0 * H:1 * H]
    _split_tables(y, av_ref, b_ref)


def _node_body(h_ref, u_in, agg_ref, g_ref, be_ref, Wc_ref, bc_ref,
               h_ref_o, u_ref, av_ref, b_ref):
    agg = jnp.concatenate([agg_ref[0], agg_ref[1]], axis=-1)
    hn = h_ref[...] + _ln_relu(u_in[...] + agg, g_ref[...], be_ref[...])
    y = hn @ Wc_ref[...] + bc_ref[...]
    h_ref_o[...] = hn
    u_ref[...] = y[:, 0 * H:1 * H]
    _split_tables(y, av_ref, b_ref)


def _node_last_body(h_ref, u_in, agg_ref, g_ref, be_ref, h_ref_o):
    agg = jnp.concatenate([agg_ref[0], agg_ref[1]], axis=-1)
    h_ref_o[...] = h_ref[...] + _ln_relu(u_in[...] + agg,
                                         g_ref[...], be_ref[...])


def _edge0_body(e_ref, eW_ref, eb_ref, Wc_ref, bc_ref, f_ref, ce_ref):
    f = e_ref[...] @ eW_ref[...] + eb_ref[...]
    ce = f @ Wc_ref[...] + bc_ref[...]
    f_ref[...] = f
    ce_ref[0] = ce[:, :HH].astype(_bf16)
    ce_ref[1] = ce[:, HH:].astype(_bf16)


def _edge_body(f_ref, en_ref, g_ref, be_ref, Wc_ref, bc_ref, f_o, ce_ref):
    en = jnp.concatenate([en_ref[0], en_ref[1]], axis=-1).astype(_f32)
    fn = f_ref[...] + _ln_relu(en, g_ref[...], be_ref[...])
    ce = fn @ Wc_ref[...] + bc_ref[...]
    f_o[...] = fn
    ce_ref[0] = ce[:, :HH].astype(_bf16)
    ce_ref[1] = ce[:, HH:].astype(_bf16)


def _edge_last_body(f_ref, en_ref, g_ref, be_ref, oW_ref, ob_ref, out_ref):
    en = jnp.concatenate([en_ref[0], en_ref[1]], axis=-1).astype(_f32)
    fn = f_ref[...] + _ln_relu(en, g_ref[...], be_ref[...])
    out_ref[...] = fn @ oW_ref[...] + ob_ref[...]


def _rows(bs):
    return pl.BlockSpec(bs, lambda r: (r, 0))


def _srows(rows, w=HH):
    return pl.BlockSpec((2, rows, w), lambda r: (0, r, 0))


def _full(shape):
    return pl.BlockSpec(shape, lambda r: tuple(0 for _ in shape))


_NODE_OUT_SHAPE = [jax.ShapeDtypeStruct((N, H), _f32)] * 2 + \
                  [jax.ShapeDtypeStruct((2, N_PAD, 2 * HH), _bf16),
                   jax.ShapeDtypeStruct((2, N_PAD, HH), _f32)]
_NODE_OUT_SPECS = [_rows((RN, H))] * 2 + [_srows(RN, 2 * HH), _srows(RN)]


def _node0(x, nW, nb, Wcat, bcat):
    return pl.pallas_call(
        _node0_body,
        grid=(N // RN,),
        in_specs=[_rows((RN, 2)), _full((2, H)), _full((1, H)),
                  _full((H, 4 * H)), _full((1, 4 * H))],
        out_specs=_NODE_OUT_SPECS,
        out_shape=_NODE_OUT_SHAPE,
    )(x, nW, nb, Wcat, bcat)


def _node(h, u, agg, g, be, Wcat, bcat):
    return pl.pallas_call(
        _node_body,
        grid=(N // RN,),
        in_specs=[_rows((RN, H)), _rows((RN, H)), _srows(RN)] +
                 [_full((1, H))] * 2 +
                 [_full((H, 4 * H)), _full((1, 4 * H))],
        out_specs=_NODE_OUT_SPECS,
        out_shape=_NODE_OUT_SHAPE,
    )(h, u, agg, g, be, Wcat, bcat)


def _node_last(h, u, agg, g, be):
    return pl.pallas_call(
        _node_last_body,
        grid=(N // RN,),
        in_specs=[_rows((RN, H)), _rows((RN, H)), _srows(RN)] +
                 [_full((1, H))] * 2,
        out_specs=[_rows((RN, H))],
        out_shape=[jax.ShapeDtypeStruct((N, H), _f32)],
    )(h, u, agg, g, be)[0]


def _edge0(e2, eW, eb, Wc, bc):
    return pl.pallas_call(
        _edge0_body,
        grid=(E // RE,),
        in_specs=[_rows((RE, 1)), _full((1, H)), _full((1, H)),
                  _full((H, H)), _full((1, H))],
        out_specs=[_rows((RE, H)), _srows(RE)],
        out_shape=[jax.ShapeDtypeStruct((E, H), _f32),
                   jax.ShapeDtypeStruct((2, E, HH), _bf16)],
    )(e2, eW, eb, Wc, bc)


def _edge(f, en, g, be, Wc, bc):
    return pl.pallas_call(
        _edge_body,
        grid=(E // RE,),
        in_specs=[_rows((RE, H)), _srows(RE)] + [_full((1, H))] * 2 +
                 [_full((H, H)), _full((1, H))],
        out_specs=[_rows((RE, H)), _srows(RE)],
        out_shape=[jax.ShapeDtypeStruct((E, H), _f32),
                   jax.ShapeDtypeStruct((2, E, HH), _bf16)],
    )(f, en, g, be, Wc, bc)


def _edge_last(f, en, g, be, oW, ob):
    return pl.pallas_call(
        _edge_last_body,
        grid=(E // RE,),
        in_specs=[_rows((RE, H)), _srows(RE)] + [_full((1, H))] * 2 +
                 [_full((H, 2)), _full((1, 2))],
        out_specs=[_rows((RE, 2))],
        out_shape=[jax.ShapeDtypeStruct((E, 2), _f32)],
    )(f, en, g, be, oW, ob)[0]


_HI_MASK = -65536


def _unpack_lo(w):
    return lax.bitcast_convert_type(jnp.left_shift(w, 16), _f32)


def _unpack_hi(w):
    return lax.bitcast_convert_type(jnp.bitwise_and(w, _HI_MASK), _f32)


def _pack_rn(e0, e1):
    r0 = lax.shift_right_logical(lax.bitcast_convert_type(e0, _i32) + 32768, 16)
    r1 = jnp.bitwise_and(lax.bitcast_convert_type(e1, _i32) + 32768, _HI_MASK)
    return jnp.bitwise_or(r0, r1)


def _sc_edge_body(av2, b2, ce2, src1d, dst1d, zrows, en2, agg2,
                  src_sb, dst_sb, dst_aj, didx2,
                  avbuf, bbuf, ebuf, mbuf,
                  sem_av, sem_b, sem_ce, sem_e, sem_m, agg_sh):
    c = lax.axis_index("c")
    s = lax.axis_index("s")
    coff_n = c * N_PAD
    ebase = s * EPS
    cebase = c * (E // 2) + s * (EPS // 2)

    pltpu.sync_copy(zrows, agg_sh.at[pl.ds(s * NPS, NPS)])
    plsc.subcore_barrier()

    def stage(bi):
        pb = (bi % 2) * SBE
        eoff = ebase + bi * SBE
        pltpu.sync_copy(src1d.at[pl.ds(eoff, SBE)],
                        src_sb.at[pl.ds(pb, SBE)])
        pltpu.sync_copy(dst1d.at[pl.ds(eoff, SBE)],
                        dst_sb.at[pl.ds(pb, SBE)])

        @plsc.parallel_loop(0, SBE // 16, unroll=5)
        def adj(k):
            w = pb + k * 16
            src_sb[pl.ds(w, 16)] = src_sb[pl.ds(w, 16)] + coff_n
            dst_aj[pl.ds(w, 16)] = dst_sb[pl.ds(w, 16)] + coff_n

    def issue(q):
        p = (q % 2) * CE
        w = ((q // SBC) % 2) * SBE + (q % SBC) * CE
        pltpu.async_copy(av2.at[src_sb.at[pl.ds(w, CE)]],
                         avbuf.at[pl.ds(p, CE)], sem_av.at[q % 2])
        pltpu.async_copy(b2.at[dst_aj.at[pl.ds(w, CE)]],
                         bbuf.at[pl.ds(p, CE)], sem_b.at[q % 2])
        pltpu.async_copy(ce2.at[pl.ds(cebase + q * CER, CER)],
                         ebuf.at[pl.ds((q % 2) * CER, CER)], sem_ce.at[q % 2])

    stage(0)
    issue(0)

    def body(j, _):
        nxt = j + 1
        jp = j % 2
        np_ = nxt % 2
        p = jp * CE
        p16 = jp * CER

        @pl.when(jnp.logical_and(nxt % SBC == 0, nxt < CHUNKS))
        def _():
            stage(nxt // SBC)

        @pl.when(j >= 1)
        def _():
            pltpu.make_async_copy(ebuf.at[pl.ds(np_ * CER, CER)],
                                  en2.at[pl.ds(0, CER)], sem_e.at[np_]).wait()

        @pl.when(nxt < CHUNKS)
        def _():
            issue(nxt)

        pltpu.make_async_copy(av2.at[pl.ds(0, CE)],
                              avbuf.at[pl.ds(p, CE)], sem_av.at[jp]).wait()
        pltpu.make_async_copy(b2.at[pl.ds(0, CE)],
                              bbuf.at[pl.ds(p, CE)], sem_b.at[jp]).wait()
        pltpu.make_async_copy(ce2.at[pl.ds(0, CER)],
                              ebuf.at[pl.ds(p16, CER)], sem_ce.at[jp]).wait()

        @pl.when(j >= 2)
        def _():
            pltpu.make_async_copy(mbuf.at[pl.ds(p, CE)],
                                  agg_sh.at[pl.ds(0, CE)], sem_m.at[jp]).wait()

        @plsc.parallel_loop(0, CE * 4, unroll=8)
        def compute(k):
            r = k // 4
            g4 = (k % 4) * 16
            row = p + r
            erow = p16 + k // 8
            ecol = ((k // 4) % 2) * 64 + g4
            aw = avbuf[row, pl.ds(g4, 16)]
            vw = avbuf[row, pl.ds(64 + g4, 16)]
            b0 = bbuf[row, pl.ds(g4 * 2, 16)]
            b1 = bbuf[row, pl.ds(g4 * 2 + 16, 16)]
            cw = ebuf[erow, pl.ds(ecol, 16)]
            e0 = _unpack_lo(aw) + b0 + _unpack_lo(cw)
            e1 = _unpack_hi(aw) + b1 + _unpack_hi(cw)
            g0 = 1.0 / (1.0 + jnp.exp(-e0))
            g1 = 1.0 / (1.0 + jnp.exp(-e1))
            mbuf[row, pl.ds(g4 * 2, 16)] = g0 * _unpack_lo(vw)
            mbuf[row, pl.ds(g4 * 2 + 16, 16)] = g1 * _unpack_hi(vw)
            ebuf[erow, pl.ds(ecol, 16)] = _pack_rn(e0, e1)

        w = ((j // SBC) % 2) * SBE + (j % SBC) * CE
        didx2[jp, pl.ds(0, 16)] = dst_sb[pl.ds(w, 16)]
        didx2[jp, pl.ds(16, 16)] = dst_sb[pl.ds(w + 16, 16)]
        pltpu.async_copy(ebuf.at[pl.ds(p16, CER)],
                         en2.at[pl.ds(cebase + j * CER, CER)], sem_e.at[jp])
        pltpu.async_copy(mbuf.at[pl.ds(p, CE)],
                         agg_sh.at[didx2.at[jp]], sem_m.at[jp], add=True)
        return 0

    lax.fori_loop(0, CHUNKS, body, 0)
    pltpu.make_async_copy(ebuf.at[pl.ds(0, CER)],
                          en2.at[pl.ds(0, CER)], sem_e.at[0]).wait()
    for jp in range(2):
        pltpu.make_async_copy(mbuf.at[pl.ds(jp * CE, CE)],
                              agg_sh.at[pl.ds(0, CE)], sem_m.at[jp]).wait()
    plsc.subcore_barrier()
    pltpu.sync_copy(agg_sh.at[pl.ds(s * NPS, NPS)],
                    agg2.at[pl.ds(c * N_PAD + s * NPS, NPS)])


def _sc_edge(av2, b2, ce2, src1d, dst1d, zrows):
    mesh = plsc.VectorSubcoreMesh(core_axis_name="c", subcore_axis_name="s",
                                  num_cores=2, num_subcores=NSUB)
    return pl.kernel(
        _sc_edge_body,
        out_type=[jax.ShapeDtypeStruct((E, 2 * HH // 2), _i32),
                  jax.ShapeDtypeStruct((2 * N_PAD, HH), _f32)],
        mesh=mesh,
        scratch_types=[
            pltpu.VMEM((2 * SBE,), _i32),
            pltpu.VMEM((2 * SBE,), _i32),
            pltpu.VMEM((2 * SBE,), _i32),
            pltpu.VMEM((2, CE), _i32),
            pltpu.VMEM((2 * CE, HH), _i32),
            pltpu.VMEM((2 * CE, HH), _f32),
            pltpu.VMEM((2 * CER, 2 * HH // 2), _i32),
            pltpu.VMEM((2 * CE, HH), _f32),
            pltpu.SemaphoreType.DMA((2,)),
            pltpu.SemaphoreType.DMA((2,)),
            pltpu.SemaphoreType.DMA((2,)),
            pltpu.SemaphoreType.DMA((2,)),
            pltpu.SemaphoreType.DMA((2,)),
            pltpu.VMEM_SHARED((N_PAD, HH), _f32),
        ],
    )(av2, b2, ce2, src1d, dst1d, zrows)


def _to_i32(bf_arr):
    s = bf_arr.shape
    return lax.bitcast_convert_type(
        bf_arr.reshape(s[:-1] + (s[-1] // 2, 2)), _i32)


def _to_bf16(i_arr):
    out = lax.bitcast_convert_type(i_arr, _bf16)
    s = out.shape
    return out.reshape(s[:-2] + (s[-2] * 2,))


def kernel(x, e, edge_index, node_W, node_b, edge_W, edge_b,
           Wu, bu, Wv, bv, Wa, ba, Wb, bb, Wc, bc,
           ln_x_g, ln_x_b, ln_e_g, ln_e_b, out_W, out_b):
    src1d = edge_index[0]
    dst1d = edge_index[1]
    zrows = jnp.zeros((NPS, HH), _f32)
    P = jnp.asarray(_P)

    node_W_p = node_W[:, P]
    node_b_p = node_b[P]
    Wcat = jnp.concatenate([Wu[:, P, :][:, :, P], Wv[:, P, :],
                            Wa[:, P, :], Wb[:, P, :][:, :, P]], axis=2)
    bcat = jnp.concatenate([bu[:, P], bv, ba, bb[:, P]], axis=1)
    ln_x_g_p = ln_x_g[:, P]
    ln_x_b_p = ln_x_b[:, P]

    r1 = lambda v: v.reshape(1, -1)

    h, ux, av, bsp = _node0(x, node_W_p, r1(node_b_p), Wcat[0], r1(bcat[0]))
    f, ce = _edge0(e.reshape(E, 1), edge_W, r1(edge_b), Wc[0], r1(bc[0]))

    for i in range(L):
        en2, agg2 = _sc_edge(_to_i32(av).reshape(2 * N_PAD, HH),
                             bsp.reshape(2 * N_PAD, HH),
                             _to_i32(ce).reshape(E, HH),
                             src1d, dst1d, zrows)
        e_new = _to_bf16(en2.reshape(2, E, HH // 2))
        agg = agg2.reshape(2, N_PAD, HH)
        gx, bxx = r1(ln_x_g_p[i]), r1(ln_x_b_p[i])
        ge, bee = r1(ln_e_g[i]), r1(ln_e_b[i])
        if i < L - 1:
            h, ux, av, bsp = _node(h, ux, agg, gx, bxx,
                                   Wcat[i + 1], r1(bcat[i + 1]))
            f, ce = _edge(f, e_new, ge, bee, Wc[i + 1], r1(bc[i + 1]))
        else:
            h = _node_last(h, ux, agg, gx, bxx)
            e_out = _edge_last(f, e_new, ge, bee, out_W, r1(out_b))
    return (h[:, jnp.asarray(_INV_P)], e_out)


# speedup vs baseline: 1.0013x; 1.0013x over previous
import jax
import jax.numpy as jnp
from jax import lax
from jax.experimental import pallas as pl
from jax.experimental.pallas import tpu as pltpu
from jax.experimental.pallas import tpu_sc as plsc

N = 10000
E = 320000
H = 256
HH = H // 2
L = 12

RN = 1000
RE = 1000

NSUB = 16
EPS = E // NSUB
CE = 32
CHUNKS = EPS // CE
SBC = 25
SBE = SBC * CE
CER = CE // 2
N_PAD = 10240
NPS = N_PAD // NSUB

_f32 = jnp.float32
_bf16 = jnp.bfloat16
_i32 = jnp.int32


def _ln_relu(t, g, b):
    mu = jnp.mean(t, axis=-1, keepdims=True)
    var = jnp.mean((t - mu) ** 2, axis=-1, keepdims=True)
    return jnp.maximum((t - mu) * jax.lax.rsqrt(var + 1e-5) * g + b, 0.0)


def _split_tables(y, av_ref, b_ref):
    av_ref[0] = jnp.concatenate([y[:, 2 * H:2 * H + HH],
                                 y[:, 1 * H:1 * H + HH]], axis=-1).astype(_bf16)
    av_ref[1] = jnp.concatenate([y[:, 2 * H + HH:3 * H],
                                 y[:, 1 * H + HH:2 * H]], axis=-1).astype(_bf16)
    b_ref[0] = y[:, 3 * H:3 * H + HH]
    b_ref[1] = y[:, 3 * H + HH:4 * H]


def _node0_body(x_ref, nW_ref, nb_ref, Wc_ref, bc_ref,
                h_ref, u_ref, av_ref, b_ref):
    h = x_ref[...] @ nW_ref[...] + nb_ref[...]
    y = h @ Wc_ref[...] + bc_ref[...]
    h_ref[...] = h
    u_ref[...] = y[:, 0 * H:1 * H]
    _split_tables(y, av_ref, b_ref)


def _node_body(h_ref, u_in, agg_ref, g_ref, be_ref, Wc_ref, bc_ref,
               h_ref_o, u_ref, av_ref, b_ref):
    agg = jnp.concatenate([agg_ref[0], agg_ref[1]], axis=-1)
    hn = h_ref[...] + _ln_relu(u_in[...] + agg, g_ref[...], be_ref[...])
    y = hn @ Wc_ref[...] + bc_ref[...]
    h_ref_o[...] = hn
    u_ref[...] = y[:, 0 * H:1 * H]
    _split_tables(y, av_ref, b_ref)


def _node_last_body(h_ref, u_in, agg_ref, g_ref, be_ref, h_ref_o):
    agg = jnp.concatenate([agg_ref[0], agg_ref[1]], axis=-1)
    h_ref_o[...] = h_ref[...] + _ln_relu(u_in[...] + agg,
                                         g_ref[...], be_ref[...])


def _edge0_body(e_ref, eW_ref, eb_ref, Wc_ref, bc_ref, f_ref, ce_ref):
    f = e_ref[...] @ eW_ref[...] + eb_ref[...]
    ce = f @ Wc_ref[...] + bc_ref[...]
    f_ref[...] = f
    ce_ref[0] = ce[:, :HH].astype(_bf16)
    ce_ref[1] = ce[:, HH:].astype(_bf16)


def _edge_body(f_ref, en_ref, g_ref, be_ref, Wc_ref, bc_ref, f_o, ce_ref):
    en = jnp.concatenate([en_ref[0], en_ref[1]], axis=-1).astype(_f32)
    fn = f_ref[...] + _ln_relu(en, g_ref[...], be_ref[...])
    ce = fn @ Wc_ref[...] + bc_ref[...]
    f_o[...] = fn
    ce_ref[0] = ce[:, :HH].astype(_bf16)
    ce_ref[1] = ce[:, HH:].astype(_bf16)


def _edge_last_body(f_ref, en_ref, g_ref, be_ref, oW_ref, ob_ref, out_ref):
    en = jnp.concatenate([en_ref[0], en_ref[1]], axis=-1).astype(_f32)
    fn = f_ref[...] + _ln_relu(en, g_ref[...], be_ref[...])
    out_ref[...] = fn @ oW_ref[...] + ob_ref[...]


def _rows(bs):
    return pl.BlockSpec(bs, lambda r: (r, 0))


def _srows(rows, w=HH):
    return pl.BlockSpec((2, rows, w), lambda r: (0, r, 0))


def _full(shape):
    return pl.BlockSpec(shape, lambda r: tuple(0 for _ in shape))


_NODE_OUT_SHAPE = [jax.ShapeDtypeStruct((N, H), _f32)] * 2 + \
                  [jax.ShapeDtypeStruct((2, N_PAD, 2 * HH), _bf16),
                   jax.ShapeDtypeStruct((2, N_PAD, HH), _f32)]
_NODE_OUT_SPECS = [_rows((RN, H))] * 2 + [_srows(RN, 2 * HH), _srows(RN)]


def _node0(x, nW, nb, Wcat, bcat):
    return pl.pallas_call(
        _node0_body,
        grid=(N // RN,),
        in_specs=[_rows((RN, 2)), _full((2, H)), _full((1, H)),
                  _full((H, 4 * H)), _full((1, 4 * H))],
        out_specs=_NODE_OUT_SPECS,
        out_shape=_NODE_OUT_SHAPE,
    )(x, nW, nb, Wcat, bcat)


def _node(h, u, agg, g, be, Wcat, bcat):
    return pl.pallas_call(
        _node_body,
        grid=(N // RN,),
        in_specs=[_rows((RN, H)), _rows((RN, H)), _srows(RN)] +
                 [_full((1, H))] * 2 +
                 [_full((H, 4 * H)), _full((1, 4 * H))],
        out_specs=_NODE_OUT_SPECS,
        out_shape=_NODE_OUT_SHAPE,
    )(h, u, agg, g, be, Wcat, bcat)


def _node_last(h, u, agg, g, be):
    return pl.pallas_call(
        _node_last_body,
        grid=(N // RN,),
        in_specs=[_rows((RN, H)), _rows((RN, H)), _srows(RN)] +
                 [_full((1, H))] * 2,
        out_specs=[_rows((RN, H))],
        out_shape=[jax.ShapeDtypeStruct((N, H), _f32)],
    )(h, u, agg, g, be)[0]


def _edge0(e2, eW, eb, Wc, bc):
    return pl.pallas_call(
        _edge0_body,
        grid=(E // RE,),
        in_specs=[_rows((RE, 1)), _full((1, H)), _full((1, H)),
                  _full((H, H)), _full((1, H))],
        out_specs=[_rows((RE, H)), _srows(RE)],
        out_shape=[jax.ShapeDtypeStruct((E, H), _f32),
                   jax.ShapeDtypeStruct((2, E, HH), _bf16)],
    )(e2, eW, eb, Wc, bc)


def _edge(f, en, g, be, Wc, bc):
    return pl.pallas_call(
        _edge_body,
        grid=(E // RE,),
        in_specs=[_rows((RE, H)), _srows(RE)] + [_full((1, H))] * 2 +
                 [_full((H, H)), _full((1, H))],
        out_specs=[_rows((RE, H)), _srows(RE)],
        out_shape=[jax.ShapeDtypeStruct((E, H), _f32),
                   jax.ShapeDtypeStruct((2, E, HH), _bf16)],
    )(f, en, g, be, Wc, bc)


def _edge_last(f, en, g, be, oW, ob):
    return pl.pallas_call(
        _edge_last_body,
        grid=(E // RE,),
        in_specs=[_rows((RE, H)), _srows(RE)] + [_full((1, H))] * 2 +
                 [_full((H, 2)), _full((1, 2))],
        out_specs=[_rows((RE, 2))],
        out_shape=[jax.ShapeDtypeStruct((E, 2), _f32)],
    )(f, en, g, be, oW, ob)[0]


_HI_MASK = -65536


def _unpack_lo(w):
    return lax.bitcast_convert_type(jnp.left_shift(w, 16), _f32)


def _unpack_hi(w):
    return lax.bitcast_convert_type(jnp.bitwise_and(w, _HI_MASK), _f32)


def _pack_rn(e0, e1):
    r0 = lax.shift_right_logical(lax.bitcast_convert_type(e0, _i32) + 32768, 16)
    r1 = jnp.bitwise_and(lax.bitcast_convert_type(e1, _i32) + 32768, _HI_MASK)
    return jnp.bitwise_or(r0, r1)


def _sc_edge_body(av2, b2, ce2, src1d, dst1d, zrows, en2, agg2,
                  src_sb, dst_sb, dst_aj, didx2,
                  avbuf, bbuf, ebuf, mbuf,
                  sem_av, sem_b, sem_ce, sem_e, sem_m, agg_sh):
    c = lax.axis_index("c")
    s = lax.axis_index("s")
    coff_n = c * N_PAD
    ebase = s * EPS
    cebase = c * (E // 2) + s * (EPS // 2)

    pltpu.sync_copy(zrows, agg_sh.at[pl.ds(s * NPS, NPS)])
    plsc.subcore_barrier()

    def stage(bi):
        pb = (bi % 2) * SBE
        eoff = ebase + bi * SBE
        pltpu.sync_copy(src1d.at[pl.ds(eoff, SBE)],
                        src_sb.at[pl.ds(pb, SBE)])
        pltpu.sync_copy(dst1d.at[pl.ds(eoff, SBE)],
                        dst_sb.at[pl.ds(pb, SBE)])

        @plsc.parallel_loop(0, SBE // 16, unroll=5)
        def adj(k):
            w = pb + k * 16
            src_sb[pl.ds(w, 16)] = src_sb[pl.ds(w, 16)] + coff_n
            dst_aj[pl.ds(w, 16)] = dst_sb[pl.ds(w, 16)] + coff_n

    def issue(q):
        p = (q % 2) * CE
        w = ((q // SBC) % 2) * SBE + (q % SBC) * CE
        pltpu.async_copy(av2.at[src_sb.at[pl.ds(w, CE)]],
                         avbuf.at[pl.ds(p, CE)], sem_av.at[q % 2])
        pltpu.async_copy(b2.at[dst_aj.at[pl.ds(w, CE)]],
                         bbuf.at[pl.ds(p, CE)], sem_b.at[q % 2])
        pltpu.async_copy(ce2.at[pl.ds(cebase + q * CER, CER)],
                         ebuf.at[pl.ds((q % 2) * CER, CER)], sem_ce.at[q % 2])

    stage(0)
    issue(0)

    def body(j, _):
        nxt = j + 1
        jp = j % 2
        np_ = nxt % 2
        p = jp * CE
        p16 = jp * CER

        @pl.when(jnp.logical_and(nxt % SBC == 0, nxt < CHUNKS))
        def _():
            stage(nxt // SBC)

        @pl.when(j >= 1)
        def _():
            pltpu.make_async_copy(ebuf.at[pl.ds(np_ * CER, CER)],
                                  en2.at[pl.ds(0, CER)], sem_e.at[np_]).wait()

        @pl.when(nxt < CHUNKS)
        def _():
            issue(nxt)

        pltpu.make_async_copy(av2.at[pl.ds(0, CE)],
                              avbuf.at[pl.ds(p, CE)], sem_av.at[jp]).wait()
        pltpu.make_async_copy(b2.at[pl.ds(0, CE)],
                              bbuf.at[pl.ds(p, CE)], sem_b.at[jp]).wait()
        pltpu.make_async_copy(ce2.at[pl.ds(0, CER)],
                              ebuf.at[pl.ds(p16, CER)], sem_ce.at[jp]).wait()

        @pl.when(j >= 2)
        def _():
            pltpu.make_async_copy(mbuf.at[pl.ds(p, CE)],
                                  agg_sh.at[pl.ds(0, CE)], sem_m.at[jp]).wait()

        @plsc.parallel_loop(0, CE * 4, unroll=8)
        def compute(k):
            r = k // 4
            g4 = (k % 4) * 16
            row = p + r
            erow = p16 + k // 8
            ecol = ((k // 4) % 2) * 64 + g4
            aw = avbuf[row, pl.ds(g4, 16)]
            vw = avbuf[row, pl.ds(64 + g4, 16)]
            b0 = bbuf[row, pl.ds(g4 * 2, 16)]
            b1 = bbuf[row, pl.ds(g4 * 2 + 16, 16)]
            cw = ebuf[erow, pl.ds(ecol, 16)]
            e0 = _unpack_lo(aw) + b0 + _unpack_lo(cw)
            e1 = _unpack_hi(aw) + b1 + _unpack_hi(cw)
            g0 = 1.0 / (1.0 + jnp.exp(-e0))
            g1 = 1.0 / (1.0 + jnp.exp(-e1))
            mbuf[row, pl.ds(g4 * 2, 16)] = g0 * _unpack_lo(vw)
            mbuf[row, pl.ds(g4 * 2 + 16, 16)] = g1 * _unpack_hi(vw)
            ebuf[erow, pl.ds(ecol, 16)] = _pack_rn(e0, e1)

        w = ((j // SBC) % 2) * SBE + (j % SBC) * CE
        didx2[jp, pl.ds(0, 16)] = dst_sb[pl.ds(w, 16)]
        didx2[jp, pl.ds(16, 16)] = dst_sb[pl.ds(w + 16, 16)]
        pltpu.async_copy(ebuf.at[pl.ds(p16, CER)],
                         en2.at[pl.ds(cebase + j * CER, CER)], sem_e.at[jp])
        pltpu.async_copy(mbuf.at[pl.ds(p, CE)],
                         agg_sh.at[didx2.at[jp]], sem_m.at[jp], add=True)
        return 0

    lax.fori_loop(0, CHUNKS, body, 0)
    pltpu.make_async_copy(ebuf.at[pl.ds(0, CER)],
                          en2.at[pl.ds(0, CER)], sem_e.at[0]).wait()
    for jp in range(2):
        pltpu.make_async_copy(mbuf.at[pl.ds(jp * CE, CE)],
                              agg_sh.at[pl.ds(0, CE)], sem_m.at[jp]).wait()
    plsc.subcore_barrier()
    pltpu.sync_copy(agg_sh.at[pl.ds(s * NPS, NPS)],
                    agg2.at[pl.ds(c * N_PAD + s * NPS, NPS)])


def _sc_edge(av2, b2, ce2, src1d, dst1d, zrows):
    mesh = plsc.VectorSubcoreMesh(core_axis_name="c", subcore_axis_name="s",
                                  num_cores=2, num_subcores=NSUB)
    return pl.kernel(
        _sc_edge_body,
        out_type=[jax.ShapeDtypeStruct((E, 2 * HH // 2), _i32),
                  jax.ShapeDtypeStruct((2 * N_PAD, HH), _f32)],
        mesh=mesh,
        scratch_types=[
            pltpu.VMEM((2 * SBE,), _i32),
            pltpu.VMEM((2 * SBE,), _i32),
            pltpu.VMEM((2 * SBE,), _i32),
            pltpu.VMEM((2, CE), _i32),
            pltpu.VMEM((2 * CE, HH), _i32),
            pltpu.VMEM((2 * CE, HH), _f32),
            pltpu.VMEM((2 * CER, 2 * HH // 2), _i32),
            pltpu.VMEM((2 * CE, HH), _f32),
            pltpu.SemaphoreType.DMA((2,)),
            pltpu.SemaphoreType.DMA((2,)),
            pltpu.SemaphoreType.DMA((2,)),
            pltpu.SemaphoreType.DMA((2,)),
            pltpu.SemaphoreType.DMA((2,)),
            pltpu.VMEM_SHARED((N_PAD, HH), _f32),
        ],
    )(av2, b2, ce2, src1d, dst1d, zrows)


def _to_i32(bf_arr):
    s = bf_arr.shape
    return lax.bitcast_convert_type(
        bf_arr.reshape(s[:-1] + (s[-1] // 2, 2)), _i32)


def _to_bf16(i_arr):
    out = lax.bitcast_convert_type(i_arr, _bf16)
    s = out.shape
    return out.reshape(s[:-2] + (s[-2] * 2,))


def _p_last(t):
    s = t.shape[:-1]
    return t.reshape(s + (H // 32, 16, 2)).swapaxes(-1, -2).reshape(s + (H,))


def _pinv_last(t):
    s = t.shape[:-1]
    return t.reshape(s + (H // 32, 2, 16)).swapaxes(-1, -2).reshape(s + (H,))


def _p_axis1(t):
    return jnp.moveaxis(_p_last(jnp.moveaxis(t, 1, -1)), -1, 1)


def kernel(x, e, edge_index, node_W, node_b, edge_W, edge_b,
           Wu, bu, Wv, bv, Wa, ba, Wb, bb, Wc, bc,
           ln_x_g, ln_x_b, ln_e_g, ln_e_b, out_W, out_b):
    src1d = edge_index[0]
    dst1d = edge_index[1]
    zrows = jnp.zeros((NPS, HH), _f32)

    node_W_p = _p_last(node_W)
    node_b_p = _p_last(node_b)
    Wcat = jnp.concatenate([_p_last(_p_axis1(Wu)), _p_axis1(Wv),
                            _p_axis1(Wa), _p_last(_p_axis1(Wb))], axis=2)
    bcat = jnp.concatenate([_p_last(bu), bv, ba, _p_last(bb)], axis=1)
    ln_x_g_p = _p_last(ln_x_g)
    ln_x_b_p = _p_last(ln_x_b)

    r1 = lambda v: v.reshape(1, -1)

    h, ux, av, bsp = _node0(x, node_W_p, r1(node_b_p), Wcat[0], r1(bcat[0]))
    f, ce = _edge0(e.reshape(E, 1), edge_W, r1(edge_b), Wc[0], r1(bc[0]))

    for i in range(L):
        en2, agg2 = _sc_edge(_to_i32(av).reshape(2 * N_PAD, HH),
                             bsp.reshape(2 * N_PAD, HH),
                             _to_i32(ce).reshape(E, HH),
                             src1d, dst1d, zrows)
        e_new = _to_bf16(en2.reshape(2, E, HH // 2))
        agg = agg2.reshape(2, N_PAD, HH)
        gx, bxx = r1(ln_x_g_p[i]), r1(ln_x_b_p[i])
        ge, bee = r1(ln_e_g[i]), r1(ln_e_b[i])
        if i < L - 1:
            h, ux, av, bsp = _node(h, ux, agg, gx, bxx,
                                   Wcat[i + 1], r1(bcat[i + 1]))
            f, ce = _edge(f, e_new, ge, bee, Wc[i + 1], r1(bc[i + 1]))
        else:
            h = _node_last(h, ux, agg, gx, bxx)
            e_out = _edge_last(f, e_new, ge, bee, out_W, r1(out_b))
    return (_pinv_last(h), e_out)


# speedup vs baseline: 2.7072x; 2.7037x over previous
import jax
import jax.numpy as jnp
from jax import lax
from jax.experimental import pallas as pl
from jax.experimental.pallas import tpu as pltpu
from jax.experimental.pallas import tpu_sc as plsc

N = 10000
E = 320000
H = 256
HH = H // 2
L = 12

RN = 1000
RE = 1000

NSUB = 16
EPS = E // NSUB
CE = 32
CHUNKS = EPS // CE
SBC = 25
SBE = SBC * CE
N_PAD = 10240
NPS = N_PAD // NSUB

_f32 = jnp.float32
_bf16 = jnp.bfloat16
_i32 = jnp.int32
_HI_MASK = -65536


def _ln_relu(t, g, b):
    mu = jnp.mean(t, axis=-1, keepdims=True)
    var = jnp.mean((t - mu) ** 2, axis=-1, keepdims=True)
    return jnp.maximum((t - mu) * jax.lax.rsqrt(var + 1e-5) * g + b, 0.0)


def _pack_av(a, v):
    ai = lax.bitcast_convert_type(a.astype(_bf16).astype(_f32), _i32)
    vi = lax.bitcast_convert_type(v.astype(_bf16).astype(_f32), _i32)
    return jnp.bitwise_or(lax.shift_right_logical(ai, 16),
                          jnp.bitwise_and(vi, _HI_MASK))


def _split_tables(y, av_ref, b_ref):
    av_ref[0] = _pack_av(y[:, 2 * H:2 * H + HH], y[:, 1 * H:1 * H + HH])
    av_ref[1] = _pack_av(y[:, 2 * H + HH:3 * H], y[:, 1 * H + HH:2 * H])
    b_ref[0] = y[:, 3 * H:3 * H + HH]
    b_ref[1] = y[:, 3 * H + HH:4 * H]


def _node0_body(x_ref, nW_ref, nb_ref, Wc_ref, bc_ref,
                h_ref, u_ref, av_ref, b_ref):
    h = x_ref[...] @ nW_ref[...] + nb_ref[...]
    y = h @ Wc_ref[...] + bc_ref[...]
    h_ref[...] = h
    u_ref[...] = y[:, 0 * H:1 * H]
    _split_tables(y, av_ref, b_ref)


def _node_body(h_ref, u_in, agg_ref, g_ref, be_ref, Wc_ref, bc_ref,
               h_ref_o, u_ref, av_ref, b_ref):
    agg = jnp.concatenate([agg_ref[0], agg_ref[1]], axis=-1)
    hn = h_ref[...] + _ln_relu(u_in[...] + agg, g_ref[...], be_ref[...])
    y = hn @ Wc_ref[...] + bc_ref[...]
    h_ref_o[...] = hn
    u_ref[...] = y[:, 0 * H:1 * H]
    _split_tables(y, av_ref, b_ref)


def _node_last_body(h_ref, u_in, agg_ref, g_ref, be_ref, h_ref_o):
    agg = jnp.concatenate([agg_ref[0], agg_ref[1]], axis=-1)
    h_ref_o[...] = h_ref[...] + _ln_relu(u_in[...] + agg,
                                         g_ref[...], be_ref[...])


def _edge0_body(e_ref, eW_ref, eb_ref, Wc_ref, bc_ref, f_ref, ce_ref):
    f = e_ref[...] @ eW_ref[...] + eb_ref[...]
    ce = f @ Wc_ref[...] + bc_ref[...]
    f_ref[...] = f
    ce_ref[0] = ce[:, :HH]
    ce_ref[1] = ce[:, HH:]


def _edge_body(f_ref, en_ref, g_ref, be_ref, Wc_ref, bc_ref, f_o, ce_ref):
    en = jnp.concatenate([en_ref[0], en_ref[1]], axis=-1)
    fn = f_ref[...] + _ln_relu(en, g_ref[...], be_ref[...])
    ce = fn @ Wc_ref[...] + bc_ref[...]
    f_o[...] = fn
    ce_ref[0] = ce[:, :HH]
    ce_ref[1] = ce[:, HH:]


def _edge_last_body(f_ref, en_ref, g_ref, be_ref, oW_ref, ob_ref, out_ref):
    en = jnp.concatenate([en_ref[0], en_ref[1]], axis=-1)
    fn = f_ref[...] + _ln_relu(en, g_ref[...], be_ref[...])
    out_ref[...] = fn @ oW_ref[...] + ob_ref[...]


def _rows(bs):
    return pl.BlockSpec(bs, lambda r: (r, 0))


def _srows(rows):
    return pl.BlockSpec((2, rows, HH), lambda r: (0, r, 0))


def _full(shape):
    return pl.BlockSpec(shape, lambda r: tuple(0 for _ in shape))


_NODE_OUT_SHAPE = [jax.ShapeDtypeStruct((N, H), _f32)] * 2 + \
                  [jax.ShapeDtypeStruct((2, N_PAD, HH), _i32),
                   jax.ShapeDtypeStruct((2, N_PAD, HH), _f32)]
_NODE_OUT_SPECS = [_rows((RN, H))] * 2 + [_srows(RN), _srows(RN)]


def _node0(x, nW, nb, Wcat, bcat):
    return pl.pallas_call(
        _node0_body,
        grid=(N // RN,),
        in_specs=[_rows((RN, 2)), _full((2, H)), _full((1, H)),
                  _full((H, 4 * H)), _full((1, 4 * H))],
        out_specs=_NODE_OUT_SPECS,
        out_shape=_NODE_OUT_SHAPE,
    )(x, nW, nb, Wcat, bcat)


def _node(h, u, agg, g, be, Wcat, bcat):
    return pl.pallas_call(
        _node_body,
        grid=(N // RN,),
        in_specs=[_rows((RN, H)), _rows((RN, H)), _srows(RN)] +
                 [_full((1, H))] * 2 +
                 [_full((H, 4 * H)), _full((1, 4 * H))],
        out_specs=_NODE_OUT_SPECS,
        out_shape=_NODE_OUT_SHAPE,
    )(h, u, agg, g, be, Wcat, bcat)


def _node_last(h, u, agg, g, be):
    return pl.pallas_call(
        _node_last_body,
        grid=(N // RN,),
        in_specs=[_rows((RN, H)), _rows((RN, H)), _srows(RN)] +
                 [_full((1, H))] * 2,
        out_specs=[_rows((RN, H))],
        out_shape=[jax.ShapeDtypeStruct((N, H), _f32)],
    )(h, u, agg, g, be)[0]


def _edge0(e2, eW, eb, Wc, bc):
    return pl.pallas_call(
        _edge0_body,
        grid=(E // RE,),
        in_specs=[_rows((RE, 1)), _full((1, H)), _full((1, H)),
                  _full((H, H)), _full((1, H))],
        out_specs=[_rows((RE, H)), _srows(RE)],
        out_shape=[jax.ShapeDtypeStruct((E, H), _f32),
                   jax.ShapeDtypeStruct((2, E, HH), _f32)],
    )(e2, eW, eb, Wc, bc)


def _edge(f, en, g, be, Wc, bc):
    return pl.pallas_call(
        _edge_body,
        grid=(E // RE,),
        in_specs=[_rows((RE, H)), _srows(RE)] + [_full((1, H))] * 2 +
                 [_full((H, H)), _full((1, H))],
        out_specs=[_rows((RE, H)), _srows(RE)],
        out_shape=[jax.ShapeDtypeStruct((E, H), _f32),
                   jax.ShapeDtypeStruct((2, E, HH), _f32)],
    )(f, en, g, be, Wc, bc)


def _edge_last(f, en, g, be, oW, ob):
    return pl.pallas_call(
        _edge_last_body,
        grid=(E // RE,),
        in_specs=[_rows((RE, H)), _srows(RE)] + [_full((1, H))] * 2 +
                 [_full((H, 2)), _full((1, 2))],
        out_specs=[_rows((RE, 2))],
        out_shape=[jax.ShapeDtypeStruct((E, 2), _f32)],
    )(f, en, g, be, oW, ob)[0]


def _unpack_lo(w):
    return lax.bitcast_convert_type(jnp.left_shift(w, 16), _f32)


def _unpack_hi(w):
    return lax.bitcast_convert_type(jnp.bitwise_and(w, _HI_MASK), _f32)


def _sc_edge_body(av2, b2, ce2, src1d, dst1d, zrows, en2, agg2,
                  src_sb, dst_sb, dst_aj, didx2,
                  avbuf, bbuf, ebuf, mbuf,
                  sem_av, sem_b, sem_ce, sem_e, sem_m, agg_sh):
    c = lax.axis_index("c")
    s = lax.axis_index("s")
    coff_n = c * N_PAD
    ebase = s * EPS
    ceoff = c * E + ebase

    pltpu.sync_copy(zrows, agg_sh.at[pl.ds(s * NPS, NPS)])
    plsc.subcore_barrier()

    def stage(bi):
        pb = (bi % 2) * SBE
        eoff = ebase + bi * SBE
        pltpu.sync_copy(src1d.at[pl.ds(eoff, SBE)],
                        src_sb.at[pl.ds(pb, SBE)])
        pltpu.sync_copy(dst1d.at[pl.ds(eoff, SBE)],
                        dst_sb.at[pl.ds(pb, SBE)])

        @plsc.parallel_loop(0, SBE // 16, unroll=5)
        def adj(k):
            w = pb + k * 16
            src_sb[pl.ds(w, 16)] = src_sb[pl.ds(w, 16)] + coff_n
            dst_aj[pl.ds(w, 16)] = dst_sb[pl.ds(w, 16)] + coff_n

    def issue(q):
        p = (q % 2) * CE
        w = ((q // SBC) % 2) * SBE + (q % SBC) * CE
        pltpu.async_copy(av2.at[src_sb.at[pl.ds(w, CE)]],
                         avbuf.at[pl.ds(p, CE)], sem_av.at[q % 2])
        pltpu.async_copy(b2.at[dst_aj.at[pl.ds(w, CE)]],
                         bbuf.at[pl.ds(p, CE)], sem_b.at[q % 2])
        pltpu.async_copy(ce2.at[pl.ds(ceoff + q * CE, CE)],
                         ebuf.at[pl.ds(p, CE)], sem_ce.at[q % 2])

    stage(0)
    issue(0)

    def body(j, _):
        nxt = j + 1
        jp = j % 2
        np_ = nxt % 2
        p = jp * CE

        @pl.when(jnp.logical_and(nxt % SBC == 0, nxt < CHUNKS))
        def _():
            stage(nxt // SBC)

        @pl.when(j >= 1)
        def _():
            pltpu.make_async_copy(ebuf.at[pl.ds(np_ * CE, CE)],
                                  en2.at[pl.ds(0, CE)], sem_e.at[np_]).wait()

        @pl.when(nxt < CHUNKS)
        def _():
            issue(nxt)

        pltpu.make_async_copy(av2.at[pl.ds(0, CE)],
                              avbuf.at[pl.ds(p, CE)], sem_av.at[jp]).wait()
        pltpu.make_async_copy(b2.at[pl.ds(0, CE)],
                              bbuf.at[pl.ds(p, CE)], sem_b.at[jp]).wait()
        pltpu.make_async_copy(ce2.at[pl.ds(0, CE)],
                              ebuf.at[pl.ds(p, CE)], sem_ce.at[jp]).wait()

        @pl.when(j >= 2)
        def _():
            pltpu.make_async_copy(mbuf.at[pl.ds(p, CE)],
                                  agg_sh.at[pl.ds(0, CE)], sem_m.at[jp]).wait()

        @plsc.parallel_loop(0, CE * 8, unroll=8)
        def compute(k):
            row = p + k // 8
            c16 = (k % 8) * 16
            w = avbuf[row, pl.ds(c16, 16)]
            b = bbuf[row, pl.ds(c16, 16)]
            cc = ebuf[row, pl.ds(c16, 16)]
            e = _unpack_lo(w) + b + cc
            g = 1.0 / (1.0 + jnp.exp(-e))
            mbuf[row, pl.ds(c16, 16)] = g * _unpack_hi(w)
            ebuf[row, pl.ds(c16, 16)] = e

        w = ((j // SBC) % 2) * SBE + (j % SBC) * CE
        didx2[jp, pl.ds(0, 16)] = dst_sb[pl.ds(w, 16)]
        didx2[jp, pl.ds(16, 16)] = dst_sb[pl.ds(w + 16, 16)]
        pltpu.async_copy(ebuf.at[pl.ds(p, CE)],
                         en2.at[pl.ds(ceoff + j * CE, CE)], sem_e.at[jp])
        pltpu.async_copy(mbuf.at[pl.ds(p, CE)],
                         agg_sh.at[didx2.at[jp]], sem_m.at[jp], add=True)
        return 0

    lax.fori_loop(0, CHUNKS, body, 0)
    pltpu.make_async_copy(ebuf.at[pl.ds(0, CE)],
                          en2.at[pl.ds(0, CE)], sem_e.at[0]).wait()
    for jp in range(2):
        pltpu.make_async_copy(mbuf.at[pl.ds(jp * CE, CE)],
                              agg_sh.at[pl.ds(0, CE)], sem_m.at[jp]).wait()
    plsc.subcore_barrier()
    pltpu.sync_copy(agg_sh.at[pl.ds(s * NPS, NPS)],
                    agg2.at[pl.ds(coff_n + s * NPS, NPS)])


def _sc_edge(av2, b2, ce2, src1d, dst1d, zrows):
    mesh = plsc.VectorSubcoreMesh(core_axis_name="c", subcore_axis_name="s",
                                  num_cores=2, num_subcores=NSUB)
    return pl.kernel(
        _sc_edge_body,
        out_type=[jax.ShapeDtypeStruct((2 * E, HH), _f32),
                  jax.ShapeDtypeStruct((2 * N_PAD, HH), _f32)],
        mesh=mesh,
        scratch_types=[
            pltpu.VMEM((2 * SBE,), _i32),
            pltpu.VMEM((2 * SBE,), _i32),
            pltpu.VMEM((2 * SBE,), _i32),
            pltpu.VMEM((2, CE), _i32),
            pltpu.VMEM((2 * CE, HH), _i32),
            pltpu.VMEM((2 * CE, HH), _f32),
            pltpu.VMEM((2 * CE, HH), _f32),
            pltpu.VMEM((2 * CE, HH), _f32),
            pltpu.SemaphoreType.DMA((2,)),
            pltpu.SemaphoreType.DMA((2,)),
            pltpu.SemaphoreType.DMA((2,)),
            pltpu.SemaphoreType.DMA((2,)),
            pltpu.SemaphoreType.DMA((2,)),
            pltpu.VMEM_SHARED((N_PAD, HH), _f32),
        ],
    )(av2, b2, ce2, src1d, dst1d, zrows)


def kernel(x, e, edge_index, node_W, node_b, edge_W, edge_b,
           Wu, bu, Wv, bv, Wa, ba, Wb, bb, Wc, bc,
           ln_x_g, ln_x_b, ln_e_g, ln_e_b, out_W, out_b):
    src1d = edge_index[0]
    dst1d = edge_index[1]
    zrows = jnp.zeros((NPS, HH), _f32)
    Wcat = jnp.concatenate([Wu, Wv, Wa, Wb], axis=2)
    bcat = jnp.concatenate([bu, bv, ba, bb], axis=1)

    r1 = lambda v: v.reshape(1, -1)

    h, ux, av, bsp = _node0(x, node_W, r1(node_b), Wcat[0], r1(bcat[0]))
    f, ce = _edge0(e.reshape(E, 1), edge_W, r1(edge_b), Wc[0], r1(bc[0]))

    for i in range(L):
        en2, agg2 = _sc_edge(av.reshape(2 * N_PAD, HH),
                             bsp.reshape(2 * N_PAD, HH),
                             ce.reshape(2 * E, HH),
                             src1d, dst1d, zrows)
        e_new = en2.reshape(2, E, HH)
        agg = agg2.reshape(2, N_PAD, HH)
        gx, bxx = r1(ln_x_g[i]), r1(ln_x_b[i])
        ge, bee = r1(ln_e_g[i]), r1(ln_e_b[i])
        if i < L - 1:
            h, ux, av, bsp = _node(h, ux, agg, gx, bxx,
                                   Wcat[i + 1], r1(bcat[i + 1]))
            f, ce = _edge(f, e_new, ge, bee, Wc[i + 1], r1(bc[i + 1]))
        else:
            h = _node_last(h, ux, agg, gx, bxx)
            e_out = _edge_last(f, e_new, ge, bee, out_W, r1(out_b))
    return (h, e_out)
